# baseline (speedup 1.0000x reference)
import jax
import jax.numpy as jnp
from jax.experimental import pallas as pl
from jax.experimental.pallas import tpu as pltpu

_N = 10000
_D_IN = 128
_D_HID = 64
_D_OUT = 16

_BI = 400
_CK = 2048
_GI = _N // _BI
_GK = pl.cdiv(_N, _CK)
_NPAD = _GK * _CK


def _fused_kernel(x_ref, conn_ref, w1_ref, b1_ref, wg_ref, bg_ref,
                  out_ref, p_ref):
    i = pl.program_id(0)
    k = pl.program_id(1)

    @pl.when(jnp.logical_and(i == 0, k == 0))
    def _proj():
        h = jnp.dot(x_ref[...], w1_ref[...],
                    preferred_element_type=jnp.float32)
        h = jnp.maximum(h + b1_ref[...], 0.0)
        p_ref[pl.ds(0, _N), :] = jnp.dot(h, wg_ref[...],
                                         preferred_element_type=jnp.float32)
        p_ref[pl.ds(_N, _NPAD - _N), :] = jnp.zeros(
            (_NPAD - _N, _D_OUT), jnp.float32)

    c = conn_ref[...]

    @pl.when(k == _GK - 1)
    def _masked():
        col = jax.lax.broadcasted_iota(jnp.int32, (_BI, _CK), 1)
        cm = jnp.where(col < _N - k * _CK, c, 0.0)
        out_ref[...] += jnp.dot(cm, p_ref[pl.ds(k * _CK, _CK), :],
                                preferred_element_type=jnp.float32)

    @pl.when(k == 0)
    def _first():
        out_ref[...] = jnp.dot(c, p_ref[pl.ds(0, _CK), :],
                               preferred_element_type=jnp.float32) + bg_ref[...]

    @pl.when(jnp.logical_and(k > 0, k < _GK - 1))
    def _mid():
        out_ref[...] += jnp.dot(c, p_ref[pl.ds(k * _CK, _CK), :],
                                preferred_element_type=jnp.float32)


def kernel(x, conn, W1, b1, Wg, bg):
    return pl.pallas_call(
        _fused_kernel,
        grid=(_GI, _GK),
        in_specs=[
            pl.BlockSpec((_N, _D_IN), lambda i, k: (0, 0)),
            pl.BlockSpec((_BI, _CK), lambda i, k: (i, k)),
            pl.BlockSpec((_D_IN, _D_HID), lambda i, k: (0, 0)),
            pl.BlockSpec((1, _D_HID), lambda i, k: (0, 0)),
            pl.BlockSpec((_D_HID, _D_OUT), lambda i, k: (0, 0)),
            pl.BlockSpec((1, _D_OUT), lambda i, k: (0, 0)),
        ],
        out_specs=pl.BlockSpec((_BI, _D_OUT), lambda i, k: (i, 0)),
        out_shape=jax.ShapeDtypeStruct((_N, _D_OUT), jnp.float32),
        scratch_shapes=[pltpu.MemorySpace.VMEM((_NPAD, _D_OUT), jnp.float32)],
        compiler_params=pltpu.CompilerParams(
            dimension_semantics=("arbitrary", "arbitrary")),
    )(x, conn, W1, b1.reshape(1, _D_HID), Wg, bg.reshape(1, _D_OUT))
